# baseline (speedup 1.0000x reference)
import functools

import jax
import jax.numpy as jnp
from jax import lax
from jax.experimental import pallas as pl
from jax.experimental.pallas import tpu as pltpu
from jax.experimental.pallas import tpu_sc as plsc

VOCAB = 1000000
DIM = 64
BATCH = 16384

_NUM_CORES = 2
_NUM_SUBCORES = 16
_NW = _NUM_CORES * _NUM_SUBCORES
_BPW = BATCH // _NW
_LANES = 16


@functools.partial(
    pl.kernel,
    out_type=jax.ShapeDtypeStruct((BATCH,), jnp.float32),
    mesh=plsc.VectorSubcoreMesh(core_axis_name="c", subcore_axis_name="s"),
    compiler_params=pltpu.CompilerParams(
        needs_layout_passes=False, use_tc_tiling_on_sc=False
    ),
    scratch_types=[
        pltpu.VMEM((_BPW,), jnp.int32),
        pltpu.VMEM((_BPW,), jnp.int32),
        pltpu.VMEM((_BPW, DIM), jnp.float32),
        pltpu.VMEM((_BPW, DIM), jnp.float32),
        pltpu.VMEM((_BPW,), jnp.float32),
        pltpu.VMEM((_BPW,), jnp.float32),
        pltpu.VMEM((_BPW,), jnp.float32),
        pltpu.SemaphoreType.DMA,
        pltpu.SemaphoreType.DMA,
        pltpu.SemaphoreType.DMA,
        pltpu.SemaphoreType.DMA,
    ],
)
def _glove_sc(wi_hbm, wj_hbm, bi_hbm, bj_hbm, ii_hbm, jj_hbm, out_hbm,
              idx_i, idx_j, rows_i, rows_j, bias_i, bias_j, out_v,
              sem0, sem1, sem2, sem3):
    wid = lax.axis_index("s") * _NUM_CORES + lax.axis_index("c")
    base = wid * _BPW

    pltpu.sync_copy(ii_hbm.at[pl.ds(base, _BPW)], idx_i)
    pltpu.sync_copy(jj_hbm.at[pl.ds(base, _BPW)], idx_j)

    cp0 = pltpu.async_copy(wi_hbm.at[idx_i], rows_i, sem0)
    cp1 = pltpu.async_copy(wj_hbm.at[idx_j], rows_j, sem1)
    cp2 = pltpu.async_copy(bi_hbm.at[idx_i], bias_i, sem2)
    cp3 = pltpu.async_copy(bj_hbm.at[idx_j], bias_j, sem3)
    cp0.wait()
    cp1.wait()
    cp2.wait()
    cp3.wait()

    lane = lax.iota(jnp.int32, _LANES)

    def group_body(g, carry):
        rb = g * _LANES
        row_ids = rb + lane
        acc = bias_i[pl.ds(rb, _LANES)] + bias_j[pl.ds(rb, _LANES)]
        for d in range(DIM):
            col = jnp.full((_LANES,), d, jnp.int32)
            a = plsc.load_gather(rows_i, [row_ids, col])
            b = plsc.load_gather(rows_j, [row_ids, col])
            acc = acc + a * b
        out_v[pl.ds(rb, _LANES)] = acc
        return carry

    lax.fori_loop(0, _BPW // _LANES, group_body, 0)

    pltpu.sync_copy(out_v, out_hbm.at[pl.ds(base, _BPW)])


def kernel(i_indices, j_indices, wi, wj, bi, bj):
    return _glove_sc(
        wi,
        wj,
        bi.reshape(VOCAB),
        bj.reshape(VOCAB),
        i_indices.astype(jnp.int32),
        j_indices.astype(jnp.int32),
    )
